# baseline (speedup 1.0000x reference)
import jax
import jax.numpy as jnp
import numpy as np
from jax import lax
from jax.experimental import pallas as pl
from jax.experimental.pallas import tpu as pltpu
from jax.experimental.pallas import tpu_sc as plsc

N = 10000
E = 320000
D = 128
NC, NS = 2, 16
NW = NC * NS
EPW = E // NW
CH = 80
NCH = EPW // CH
STRIPE = 624
NB = 10
R = N // NB

_mesh = plsc.VectorSubcoreMesh(core_axis_name="c", subcore_axis_name="s")


def _worker_prologue():
    c = lax.axis_index("c")
    s = lax.axis_index("s")
    base0 = (c * NS + s) * EPW
    my = pl.multiple_of(s * STRIPE, 8)

    def stripe_copy(mk_src, mk_dst):
        @pl.when(s < NS - 1)
        def _():
            pltpu.sync_copy(mk_src(STRIPE), mk_dst(STRIPE))

        @pl.when(s == NS - 1)
        def _():
            pltpu.sync_copy(mk_src(640), mk_dst(640))

    return c, s, base0, my, stripe_copy


NBUF = 4


def _segsum_body(feats, srcv, dstv, zfull, out_p,
                 s0, s1, s2, s3, d0, d1, d2, d3, r0, r1, r2, r3,
                 f0, f1, acc,
                 is0, is1, is2, is3, gs0, gs1, gs2, gs3,
                 ds0, ds1, ds2, ds3, ss0, ss1, ss2, ss3):
    c, s, base0, my, stripe_copy = _worker_prologue()
    sbuf = (s0, s1, s2, s3)
    dbuf = (d0, d1, d2, d3)
    rbuf = (r0, r1, r2, r3)
    fbuf = (f0, f1)
    isem = (is0, is1, is2, is3)
    gsem = (gs0, gs1, gs2, gs3)
    dsem = (ds0, ds1, ds2, ds3)
    ssem = (ss0, ss1, ss2, ss3)

    stripe_copy(lambda n: zfull.at[pl.ds(my, n)],
                lambda n: acc.at[pl.ds(my, n)])
    plsc.subcore_barrier()

    def convert(rbf, fb):
        mask = jnp.int32(-65536)

        def conv(r, carry):
            for g in range(D // 32):
                v32 = rbf[r, pl.ds(16 * g, 16)]
                a = lax.bitcast_convert_type(v32 << 16, jnp.float32)
                b = lax.bitcast_convert_type(v32 & mask, jnp.float32)
                fb[r, pl.ds(32 * g, 16)] = a
                fb[r, pl.ds(32 * g + 16, 16)] = b
            return carry

        lax.fori_loop(0, CH, conv, 0)

    def run_chunks(i0, nk):
        dds, iis = [], []
        for k in range(nk):
            bi = pl.multiple_of(base0 + (i0 + k) * CH, 8)
            dds.append(pltpu.async_copy(dstv.at[pl.ds(bi, CH)],
                                        dbuf[k], dsem[k]))
            iis.append(pltpu.async_copy(srcv.at[pl.ds(bi, CH)],
                                        sbuf[k], isem[k]))
        ggs = []
        for k in range(nk):
            iis[k].wait()
            ggs.append(pltpu.async_copy(feats.at[sbuf[k]], rbuf[k], gsem[k]))
        scats = []
        for k in range(nk):
            ggs[k].wait()
            if len(scats) >= 2:
                scats[k - 2].wait()
            convert(rbuf[k], fbuf[k % 2])
            dds[k].wait()
            scats.append(pltpu.async_copy(fbuf[k % 2], acc.at[dbuf[k]],
                                          ssem[k], add=True))
        for k in range(max(0, nk - 2), nk):
            scats[k].wait()

    def body(j, carry):
        run_chunks(j * NBUF, NBUF)
        return carry

    lax.fori_loop(0, NCH // NBUF, body, 0)
    run_chunks(NCH - 1, 1)

    plsc.subcore_barrier()
    dst_row = pl.multiple_of(c * N + s * STRIPE, 8)
    stripe_copy(lambda n: acc.at[pl.ds(my, n)],
                lambda n: out_p.at[pl.ds(dst_row, n)])


_segsum = pl.kernel(
    _segsum_body,
    out_type=jax.ShapeDtypeStruct((NC * N, D), jnp.float32),
    mesh=_mesh,
    scratch_types=(
        (pltpu.VMEM((CH,), jnp.int32),) * (2 * NBUF)
        + (pltpu.VMEM((CH, D // 2), jnp.int32),) * NBUF
        + (pltpu.VMEM((CH, D), jnp.float32),) * 2
        + (pltpu.VMEM_SHARED((N, D), jnp.float32),)
        + (pltpu.SemaphoreType.DMA,) * (4 * NBUF)
    ),
    compiler_params=pltpu.CompilerParams(use_tc_tiling_on_sc=False),
)


def _degsum_body(dstv, zfull, ones_h, out_p,
                 ones_v, d0, d1, d2, d3, acc,
                 ds0, ds1, ds2, ds3, ss0, ss1, ss2, ss3):
    c, s, base0, my, stripe_copy = _worker_prologue()
    dbuf = (d0, d1, d2, d3)
    dsem = (ds0, ds1, ds2, ds3)
    ssem = (ss0, ss1, ss2, ss3)

    stripe_copy(lambda n: zfull.at[pl.ds(my, n)],
                lambda n: acc.at[pl.ds(my, n)])
    pltpu.sync_copy(ones_h, ones_v)
    plsc.subcore_barrier()

    def start(i, k):
        bi = pl.multiple_of(base0 + i * CH, 8)
        return pltpu.async_copy(dstv.at[pl.ds(bi, CH)], dbuf[k], dsem[k])

    def body(j, carry):
        i0 = j * NBUF
        descs = [start(i0 + k, k) for k in range(NBUF)]
        scats = []
        for k in range(NBUF):
            descs[k].wait()
            scats.append(pltpu.async_copy(ones_v, acc.at[dbuf[k]],
                                          ssem[k], add=True))
        for sc_ in scats:
            sc_.wait()
        return carry

    lax.fori_loop(0, NCH // NBUF, body, 0)

    dd = start(NCH - 1, 0)
    dd.wait()
    pltpu.sync_copy(ones_v, acc.at[dbuf[0]], add=True)

    plsc.subcore_barrier()
    dst_row = pl.multiple_of(c * N + s * STRIPE, 8)
    stripe_copy(lambda n: acc.at[pl.ds(my, n)],
                lambda n: out_p.at[pl.ds(dst_row, n)])


_degsum = pl.kernel(
    _degsum_body,
    out_type=jax.ShapeDtypeStruct((NC * N, D), jnp.float32),
    mesh=_mesh,
    scratch_types=(
        (pltpu.VMEM((CH, D), jnp.float32),)
        + (pltpu.VMEM((CH,), jnp.int32),) * NBUF
        + (pltpu.VMEM_SHARED((N, D), jnp.float32),)
        + (pltpu.SemaphoreType.DMA,) * (2 * NBUF)
    ),
)


def _part_specs():
    return [pl.BlockSpec((R, D), lambda i: (i, 0)),
            pl.BlockSpec((R, D), lambda i: (i + NB, 0))]


_W_SPEC = pl.BlockSpec((2 * D, D), lambda i: (0, 0))
_WO_SPEC = pl.BlockSpec((D, D), lambda i: (0, 0))
_B_SPEC = pl.BlockSpec((1, D), lambda i: (0, 0))
_ROW_SPEC = pl.BlockSpec((R, D), lambda i: (i, 0))


def _gates_body(axa, axb, aha, ahb, dga, dgb, h, wr, wu, br, bu, u_o, rh_o):
    dn = 1.0 / jnp.maximum(dga[:, :1] + dgb[:, :1], 1.0)
    ax = (axa[...] + axb[...]) * dn
    ah = (aha[...] + ahb[...]) * dn
    wr_ = wr[...]
    wu_ = wu[...]
    r = jax.nn.sigmoid(jnp.dot(ax, wr_[:D], preferred_element_type=jnp.float32)
                       + jnp.dot(ah, wr_[D:], preferred_element_type=jnp.float32)
                       + br[...])
    u = jax.nn.sigmoid(jnp.dot(ax, wu_[:D], preferred_element_type=jnp.float32)
                       + jnp.dot(ah, wu_[D:], preferred_element_type=jnp.float32)
                       + bu[...])
    u_o[...] = u
    rh_o[...] = r * h[...]


_gates_call = pl.pallas_call(
    _gates_body,
    grid=(NB,),
    in_specs=_part_specs() + _part_specs() + _part_specs()
    + [_ROW_SPEC, _W_SPEC, _W_SPEC, _B_SPEC, _B_SPEC],
    out_specs=[_ROW_SPEC, _ROW_SPEC],
    out_shape=[jax.ShapeDtypeStruct((N, D), jnp.float32),
               jax.ShapeDtypeStruct((N, D), jnp.float32)],
    compiler_params=pltpu.CompilerParams(dimension_semantics=("parallel",)),
)


def _cand_body(axa, axb, aca, acb, dga, dgb, h, u, wc, bc, x_o):
    dn = 1.0 / jnp.maximum(dga[:, :1] + dgb[:, :1], 1.0)
    ax = (axa[...] + axb[...]) * dn
    ac = (aca[...] + acb[...]) * dn
    wc_ = wc[...]
    cand = jnp.tanh(jnp.dot(ax, wc_[:D], preferred_element_type=jnp.float32)
                    + jnp.dot(ac, wc_[D:], preferred_element_type=jnp.float32)
                    + bc[...])
    uu = u[...]
    x_o[...] = uu * h[...] + (1.0 - uu) * cand


_cand_call = pl.pallas_call(
    _cand_body,
    grid=(NB,),
    in_specs=_part_specs() + _part_specs() + _part_specs()
    + [_ROW_SPEC, _ROW_SPEC, _W_SPEC, _B_SPEC],
    out_specs=_ROW_SPEC,
    out_shape=jax.ShapeDtypeStruct((N, D), jnp.float32),
    compiler_params=pltpu.CompilerParams(dimension_semantics=("parallel",)),
)


def _cand_proj_body(axa, axb, aca, acb, dga, dgb, h, u, wc, bc, wo, bo,
                    x_o, out_o):
    dn = 1.0 / jnp.maximum(dga[:, :1] + dgb[:, :1], 1.0)
    ax = (axa[...] + axb[...]) * dn
    ac = (aca[...] + acb[...]) * dn
    wc_ = wc[...]
    cand = jnp.tanh(jnp.dot(ax, wc_[:D], preferred_element_type=jnp.float32)
                    + jnp.dot(ac, wc_[D:], preferred_element_type=jnp.float32)
                    + bc[...])
    uu = u[...]
    x2 = uu * h[...] + (1.0 - uu) * cand
    x_o[...] = x2
    out_o[...] = jnp.dot(x2, wo[...], preferred_element_type=jnp.float32) + bo[...]


_cand_proj_call = pl.pallas_call(
    _cand_proj_body,
    grid=(NB,),
    in_specs=_part_specs() + _part_specs() + _part_specs()
    + [_ROW_SPEC, _ROW_SPEC, _W_SPEC, _B_SPEC, _WO_SPEC, _B_SPEC],
    out_specs=[_ROW_SPEC, _ROW_SPEC],
    out_shape=[jax.ShapeDtypeStruct((N, D), jnp.float32),
               jax.ShapeDtypeStruct((N, D), jnp.float32)],
    compiler_params=pltpu.CompilerParams(dimension_semantics=("parallel",)),
)


_PERM = np.arange(128).reshape(4, 2, 16).transpose(0, 2, 1).reshape(128)


def _shuf_bf16(a):
    ab = a[:, _PERM].astype(jnp.bfloat16)
    return jax.lax.bitcast_convert_type(ab.reshape(N, D // 2, 2), jnp.int32)


def kernel(x, edge_index, h0, h1, Wr0, br0, Wu0, bu0, Wc0, bc0,
           Wr1, br1, Wu1, bu1, Wc1, bc1, Wo, bo):
    src = edge_index[0]
    dst = edge_index[1]
    zfull = jnp.zeros((N, D), jnp.float32)
    ones_h = jnp.ones((CH, D), jnp.float32)

    br0_ = br0.reshape(1, D)
    bu0_ = bu0.reshape(1, D)
    bc0_ = bc0.reshape(1, D)
    br1_ = br1.reshape(1, D)
    bu1_ = bu1.reshape(1, D)
    bc1_ = bc1.reshape(1, D)
    bo_ = bo.reshape(1, D)

    degp = _degsum(dst, zfull, ones_h)
    xagg = _segsum(_shuf_bf16(x), src, dst, zfull)
    h0agg = _segsum(_shuf_bf16(h0), src, dst, zfull)
    u0, rh0 = _gates_call(xagg, xagg, h0agg, h0agg, degp, degp,
                          h0, Wr0, Wu0, br0_, bu0_)
    rh0agg = _segsum(_shuf_bf16(rh0), src, dst, zfull)
    x1 = _cand_call(xagg, xagg, rh0agg, rh0agg, degp, degp,
                    h0, u0, Wc0, bc0_)
    x1agg = _segsum(_shuf_bf16(x1), src, dst, zfull)
    h1agg = _segsum(_shuf_bf16(h1), src, dst, zfull)
    u1, rh1 = _gates_call(x1agg, x1agg, h1agg, h1agg, degp, degp,
                          h1, Wr1, Wu1, br1_, bu1_)
    rh1agg = _segsum(_shuf_bf16(rh1), src, dst, zfull)
    x2, out = _cand_proj_call(x1agg, x1agg, rh1agg, rh1agg, degp, degp,
                              h1, u1, Wc1, bc1_, Wo, bo_)
    return (out, x1, x2)


# speedup vs baseline: 1.6836x; 1.6836x over previous
import jax
import jax.numpy as jnp
from jax import lax
from jax.experimental import pallas as pl
from jax.experimental.pallas import tpu as pltpu
from jax.experimental.pallas import tpu_sc as plsc

N = 10000
E = 320000
D = 128
NC, NS = 2, 16
NW = NC * NS
EPW = E // NW
CH = 80
NCH = EPW // CH
STRIPE = 624
NB = 10
R = N // NB

_mesh = plsc.VectorSubcoreMesh(core_axis_name="c", subcore_axis_name="s")


def _worker_prologue():
    c = lax.axis_index("c")
    s = lax.axis_index("s")
    base0 = (c * NS + s) * EPW
    my = pl.multiple_of(s * STRIPE, 8)

    def stripe_copy(mk_src, mk_dst):
        @pl.when(s < NS - 1)
        def _():
            pltpu.sync_copy(mk_src(STRIPE), mk_dst(STRIPE))

        @pl.when(s == NS - 1)
        def _():
            pltpu.sync_copy(mk_src(640), mk_dst(640))

    return c, s, base0, my, stripe_copy


NBUF = 4


def _segsum_body(feats, srcv, dstv, zfull, out_p,
                 s0, s1, s2, s3, d0, d1, d2, d3, e0, e1, e2, e3,
                 r0, r1, r2, r3, acc,
                 is0, is1, is2, is3, gs0, gs1, gs2, gs3,
                 ds0, ds1, ds2, ds3, ss0, ss1, ss2, ss3):
    c, s, base0, my, stripe_copy = _worker_prologue()
    sbuf = (s0, s1, s2, s3)
    dbuf = (d0, d1, d2, d3)
    dscat = (e0, e1, e2, e3)
    rbuf = (r0, r1, r2, r3)
    isem = (is0, is1, is2, is3)
    gsem = (gs0, gs1, gs2, gs3)
    dsem = (ds0, ds1, ds2, ds3)
    ssem = (ss0, ss1, ss2, ss3)

    stripe_copy(lambda n: zfull.at[pl.ds(my, n)],
                lambda n: acc.at[pl.ds(my, n)])
    plsc.subcore_barrier()

    def idx_start(i, k):
        bi = pl.multiple_of(base0 + i * CH, 8)
        pltpu.async_copy(dstv.at[pl.ds(bi, CH)], dbuf[k], dsem[k])
        pltpu.async_copy(srcv.at[pl.ds(bi, CH)], sbuf[k], isem[k])

    def wait_scat(k):
        pltpu.make_async_copy(rbuf[k], acc.at[dscat[k]], ssem[k]).wait()

    def wait_idx(k):
        pltpu.make_async_copy(srcv.at[pl.ds(0, CH)], sbuf[k], isem[k]).wait()
        pltpu.make_async_copy(dstv.at[pl.ds(0, CH)], dbuf[k], dsem[k]).wait()

    def process(i, k, j, prefetch):
        @pl.when(j > 0)
        def _():
            wait_scat(k)
        wait_idx(k)
        gg = pltpu.async_copy(feats.at[sbuf[k]], rbuf[k], gsem[k])
        for q in range(CH // 16):
            dscat[k][pl.ds(q * 16, 16)] = dbuf[k][pl.ds(q * 16, 16)]
        gg.wait()
        if prefetch:
            @pl.when(i + NBUF < NCH)
            def _():
                idx_start(i + NBUF, k)
        pltpu.async_copy(rbuf[k], acc.at[dscat[k]], ssem[k], add=True)

    for k in range(NBUF):
        idx_start(k, k)

    def body(j, carry):
        for k in range(NBUF):
            process(j * NBUF + k, k, j, True)
        return carry

    lax.fori_loop(0, NCH // NBUF, body, 0)
    process(NCH - 1, 0, 1, False)
    for k in range(1, NBUF):
        wait_scat(k)
    wait_scat(0)

    plsc.subcore_barrier()
    dst_row = pl.multiple_of(c * N + s * STRIPE, 8)
    stripe_copy(lambda n: acc.at[pl.ds(my, n)],
                lambda n: out_p.at[pl.ds(dst_row, n)])


_segsum = pl.kernel(
    _segsum_body,
    out_type=jax.ShapeDtypeStruct((NC * N, D), jnp.float32),
    mesh=_mesh,
    scratch_types=(
        (pltpu.VMEM((CH,), jnp.int32),) * (3 * NBUF)
        + (pltpu.VMEM((CH, D), jnp.float32),) * NBUF
        + (pltpu.VMEM_SHARED((N, D), jnp.float32),)
        + (pltpu.SemaphoreType.DMA,) * (4 * NBUF)
    ),
)


def _degsum_body(dstv, zfull, ones_h, out_p,
                 ones_v, d0, d1, d2, d3, acc,
                 ds0, ds1, ds2, ds3, ss0, ss1, ss2, ss3):
    c, s, base0, my, stripe_copy = _worker_prologue()
    dbuf = (d0, d1, d2, d3)
    dsem = (ds0, ds1, ds2, ds3)
    ssem = (ss0, ss1, ss2, ss3)

    stripe_copy(lambda n: zfull.at[pl.ds(my, n)],
                lambda n: acc.at[pl.ds(my, n)])
    pltpu.sync_copy(ones_h, ones_v)
    plsc.subcore_barrier()

    def start(i, k):
        bi = pl.multiple_of(base0 + i * CH, 8)
        return pltpu.async_copy(dstv.at[pl.ds(bi, CH)], dbuf[k], dsem[k])

    def body(j, carry):
        i0 = j * NBUF
        descs = [start(i0 + k, k) for k in range(NBUF)]
        scats = []
        for k in range(NBUF):
            descs[k].wait()
            scats.append(pltpu.async_copy(ones_v, acc.at[dbuf[k]],
                                          ssem[k], add=True))
        for sc_ in scats:
            sc_.wait()
        return carry

    lax.fori_loop(0, NCH // NBUF, body, 0)

    dd = start(NCH - 1, 0)
    dd.wait()
    pltpu.sync_copy(ones_v, acc.at[dbuf[0]], add=True)

    plsc.subcore_barrier()
    dst_row = pl.multiple_of(c * N + s * STRIPE, 8)
    stripe_copy(lambda n: acc.at[pl.ds(my, n)],
                lambda n: out_p.at[pl.ds(dst_row, n)])


_degsum = pl.kernel(
    _degsum_body,
    out_type=jax.ShapeDtypeStruct((NC * N, D), jnp.float32),
    mesh=_mesh,
    scratch_types=(
        (pltpu.VMEM((CH, D), jnp.float32),)
        + (pltpu.VMEM((CH,), jnp.int32),) * NBUF
        + (pltpu.VMEM_SHARED((N, D), jnp.float32),)
        + (pltpu.SemaphoreType.DMA,) * (2 * NBUF)
    ),
)


def _part_specs():
    return [pl.BlockSpec((R, D), lambda i: (i, 0)),
            pl.BlockSpec((R, D), lambda i: (i + NB, 0))]


_W_SPEC = pl.BlockSpec((2 * D, D), lambda i: (0, 0))
_WO_SPEC = pl.BlockSpec((D, D), lambda i: (0, 0))
_B_SPEC = pl.BlockSpec((1, D), lambda i: (0, 0))
_ROW_SPEC = pl.BlockSpec((R, D), lambda i: (i, 0))


def _gates_body(axa, axb, aha, ahb, dga, dgb, h, wr, wu, br, bu, u_o, rh_o):
    dn = 1.0 / jnp.maximum(dga[:, :1] + dgb[:, :1], 1.0)
    ax = (axa[...] + axb[...]) * dn
    ah = (aha[...] + ahb[...]) * dn
    wr_ = wr[...]
    wu_ = wu[...]
    r = jax.nn.sigmoid(jnp.dot(ax, wr_[:D], preferred_element_type=jnp.float32)
                       + jnp.dot(ah, wr_[D:], preferred_element_type=jnp.float32)
                       + br[...])
    u = jax.nn.sigmoid(jnp.dot(ax, wu_[:D], preferred_element_type=jnp.float32)
                       + jnp.dot(ah, wu_[D:], preferred_element_type=jnp.float32)
                       + bu[...])
    u_o[...] = u
    rh_o[...] = r * h[...]


_gates_call = pl.pallas_call(
    _gates_body,
    grid=(NB,),
    in_specs=_part_specs() + _part_specs() + _part_specs()
    + [_ROW_SPEC, _W_SPEC, _W_SPEC, _B_SPEC, _B_SPEC],
    out_specs=[_ROW_SPEC, _ROW_SPEC],
    out_shape=[jax.ShapeDtypeStruct((N, D), jnp.float32),
               jax.ShapeDtypeStruct((N, D), jnp.float32)],
    compiler_params=pltpu.CompilerParams(dimension_semantics=("parallel",)),
)


def _cand_body(axa, axb, aca, acb, dga, dgb, h, u, wc, bc, x_o):
    dn = 1.0 / jnp.maximum(dga[:, :1] + dgb[:, :1], 1.0)
    ax = (axa[...] + axb[...]) * dn
    ac = (aca[...] + acb[...]) * dn
    wc_ = wc[...]
    cand = jnp.tanh(jnp.dot(ax, wc_[:D], preferred_element_type=jnp.float32)
                    + jnp.dot(ac, wc_[D:], preferred_element_type=jnp.float32)
                    + bc[...])
    uu = u[...]
    x_o[...] = uu * h[...] + (1.0 - uu) * cand


_cand_call = pl.pallas_call(
    _cand_body,
    grid=(NB,),
    in_specs=_part_specs() + _part_specs() + _part_specs()
    + [_ROW_SPEC, _ROW_SPEC, _W_SPEC, _B_SPEC],
    out_specs=_ROW_SPEC,
    out_shape=jax.ShapeDtypeStruct((N, D), jnp.float32),
    compiler_params=pltpu.CompilerParams(dimension_semantics=("parallel",)),
)


def _cand_proj_body(axa, axb, aca, acb, dga, dgb, h, u, wc, bc, wo, bo,
                    x_o, out_o):
    dn = 1.0 / jnp.maximum(dga[:, :1] + dgb[:, :1], 1.0)
    ax = (axa[...] + axb[...]) * dn
    ac = (aca[...] + acb[...]) * dn
    wc_ = wc[...]
    cand = jnp.tanh(jnp.dot(ax, wc_[:D], preferred_element_type=jnp.float32)
                    + jnp.dot(ac, wc_[D:], preferred_element_type=jnp.float32)
                    + bc[...])
    uu = u[...]
    x2 = uu * h[...] + (1.0 - uu) * cand
    x_o[...] = x2
    out_o[...] = jnp.dot(x2, wo[...], preferred_element_type=jnp.float32) + bo[...]


_cand_proj_call = pl.pallas_call(
    _cand_proj_body,
    grid=(NB,),
    in_specs=_part_specs() + _part_specs() + _part_specs()
    + [_ROW_SPEC, _ROW_SPEC, _W_SPEC, _B_SPEC, _WO_SPEC, _B_SPEC],
    out_specs=[_ROW_SPEC, _ROW_SPEC],
    out_shape=[jax.ShapeDtypeStruct((N, D), jnp.float32),
               jax.ShapeDtypeStruct((N, D), jnp.float32)],
    compiler_params=pltpu.CompilerParams(dimension_semantics=("parallel",)),
)


def kernel(x, edge_index, h0, h1, Wr0, br0, Wu0, bu0, Wc0, bc0,
           Wr1, br1, Wu1, bu1, Wc1, bc1, Wo, bo):
    src = edge_index[0]
    dst = edge_index[1]
    zfull = jnp.zeros((N, D), jnp.float32)
    ones_h = jnp.ones((CH, D), jnp.float32)

    br0_ = br0.reshape(1, D)
    bu0_ = bu0.reshape(1, D)
    bc0_ = bc0.reshape(1, D)
    br1_ = br1.reshape(1, D)
    bu1_ = bu1.reshape(1, D)
    bc1_ = bc1.reshape(1, D)
    bo_ = bo.reshape(1, D)

    degp = _degsum(dst, zfull, ones_h)
    xagg = _segsum(x, src, dst, zfull)
    h0agg = _segsum(h0, src, dst, zfull)
    u0, rh0 = _gates_call(xagg, xagg, h0agg, h0agg, degp, degp,
                          h0, Wr0, Wu0, br0_, bu0_)
    rh0agg = _segsum(rh0, src, dst, zfull)
    x1 = _cand_call(xagg, xagg, rh0agg, rh0agg, degp, degp,
                    h0, u0, Wc0, bc0_)
    x1agg = _segsum(x1, src, dst, zfull)
    h1agg = _segsum(h1, src, dst, zfull)
    u1, rh1 = _gates_call(x1agg, x1agg, h1agg, h1agg, degp, degp,
                          h1, Wr1, Wu1, br1_, bu1_)
    rh1agg = _segsum(rh1, src, dst, zfull)
    x2, out = _cand_proj_call(x1agg, x1agg, rh1agg, rh1agg, degp, degp,
                              h1, u1, Wc1, bc1_, Wo, bo_)
    return (out, x1, x2)


# speedup vs baseline: 1.7071x; 1.0140x over previous
import jax
import jax.numpy as jnp
from jax import lax
from jax.experimental import pallas as pl
from jax.experimental.pallas import tpu as pltpu
from jax.experimental.pallas import tpu_sc as plsc

N = 10000
E = 320000
D = 128
NC, NS = 2, 16
NW = NC * NS
EPW = E // NW
CH = 80
NCH = EPW // CH
STRIPE = 624
NB = 10
R = N // NB

_mesh = plsc.VectorSubcoreMesh(core_axis_name="c", subcore_axis_name="s")


def _worker_prologue():
    c = lax.axis_index("c")
    s = lax.axis_index("s")
    base0 = (c * NS + s) * EPW
    my = pl.multiple_of(s * STRIPE, 8)

    def stripe_copy(mk_src, mk_dst):
        @pl.when(s < NS - 1)
        def _():
            pltpu.sync_copy(mk_src(STRIPE), mk_dst(STRIPE))

        @pl.when(s == NS - 1)
        def _():
            pltpu.sync_copy(mk_src(640), mk_dst(640))

    return c, s, base0, my, stripe_copy


NBUF = 4


def _segsum_body(feats, srcv, dstv, zfull, out_p,
                 s0, s1, s2, s3, d0, d1, d2, d3, r0, r1, r2, r3, acc,
                 is0, is1, is2, is3, gs0, gs1, gs2, gs3,
                 ds0, ds1, ds2, ds3, ss0, ss1, ss2, ss3):
    c, s, base0, my, stripe_copy = _worker_prologue()
    sbuf = (s0, s1, s2, s3)
    dbuf = (d0, d1, d2, d3)
    rbuf = (r0, r1, r2, r3)
    isem = (is0, is1, is2, is3)
    gsem = (gs0, gs1, gs2, gs3)
    dsem = (ds0, ds1, ds2, ds3)
    ssem = (ss0, ss1, ss2, ss3)

    stripe_copy(lambda n: zfull.at[pl.ds(my, n)],
                lambda n: acc.at[pl.ds(my, n)])
    plsc.subcore_barrier()

    def run_chunks(i0, nk):
        dds, iis = [], []
        for k in range(nk):
            bi = pl.multiple_of(base0 + (i0 + k) * CH, 8)
            dds.append(pltpu.async_copy(dstv.at[pl.ds(bi, CH)],
                                        dbuf[k], dsem[k]))
            iis.append(pltpu.async_copy(srcv.at[pl.ds(bi, CH)],
                                        sbuf[k], isem[k]))
        ggs = []
        for k in range(nk):
            iis[k].wait()
            ggs.append(pltpu.async_copy(feats.at[sbuf[k]], rbuf[k], gsem[k]))
        scats = []
        for k in range(nk):
            dds[k].wait()
            ggs[k].wait()
            scats.append(pltpu.async_copy(rbuf[k], acc.at[dbuf[k]],
                                          ssem[k], add=True))
        for sc_ in scats:
            sc_.wait()

    def body(j, carry):
        run_chunks(j * NBUF, NBUF)
        return carry

    lax.fori_loop(0, NCH // NBUF, body, 0)
    run_chunks(NCH - 1, 1)

    plsc.subcore_barrier()
    dst_row = pl.multiple_of(c * N + s * STRIPE, 8)
    stripe_copy(lambda n: acc.at[pl.ds(my, n)],
                lambda n: out_p.at[pl.ds(dst_row, n)])


_segsum = pl.kernel(
    _segsum_body,
    out_type=jax.ShapeDtypeStruct((NC * N, D), jnp.float32),
    mesh=_mesh,
    scratch_types=(
        (pltpu.VMEM((CH,), jnp.int32),) * (2 * NBUF)
        + (pltpu.VMEM((CH, D), jnp.float32),) * NBUF
        + (pltpu.VMEM_SHARED((N, D), jnp.float32),)
        + (pltpu.SemaphoreType.DMA,) * (4 * NBUF)
    ),
)


def _degsum_body(dstv, zfull, ones_h, out_p,
                 ones_v, d0, d1, d2, d3, acc,
                 ds0, ds1, ds2, ds3, ss0, ss1, ss2, ss3):
    c, s, base0, my, stripe_copy = _worker_prologue()
    dbuf = (d0, d1, d2, d3)
    dsem = (ds0, ds1, ds2, ds3)
    ssem = (ss0, ss1, ss2, ss3)

    stripe_copy(lambda n: zfull.at[pl.ds(my, n)],
                lambda n: acc.at[pl.ds(my, n)])
    pltpu.sync_copy(ones_h, ones_v)
    plsc.subcore_barrier()

    def start(i, k):
        bi = pl.multiple_of(base0 + i * CH, 8)
        return pltpu.async_copy(dstv.at[pl.ds(bi, CH)], dbuf[k], dsem[k])

    def body(j, carry):
        i0 = j * NBUF
        descs = [start(i0 + k, k) for k in range(NBUF)]
        scats = []
        for k in range(NBUF):
            descs[k].wait()
            scats.append(pltpu.async_copy(ones_v, acc.at[dbuf[k]],
                                          ssem[k], add=True))
        for sc_ in scats:
            sc_.wait()
        return carry

    lax.fori_loop(0, NCH // NBUF, body, 0)

    dd = start(NCH - 1, 0)
    dd.wait()
    pltpu.sync_copy(ones_v, acc.at[dbuf[0]], add=True)

    plsc.subcore_barrier()
    dst_row = pl.multiple_of(c * N + s * STRIPE, 8)
    stripe_copy(lambda n: acc.at[pl.ds(my, n)],
                lambda n: out_p.at[pl.ds(dst_row, n)])


_degsum = pl.kernel(
    _degsum_body,
    out_type=jax.ShapeDtypeStruct((NC * N, D), jnp.float32),
    mesh=_mesh,
    scratch_types=(
        (pltpu.VMEM((CH, D), jnp.float32),)
        + (pltpu.VMEM((CH,), jnp.int32),) * NBUF
        + (pltpu.VMEM_SHARED((N, D), jnp.float32),)
        + (pltpu.SemaphoreType.DMA,) * (2 * NBUF)
    ),
)


def _part_specs():
    return [pl.BlockSpec((R, D), lambda i: (i, 0)),
            pl.BlockSpec((R, D), lambda i: (i + NB, 0))]


_W_SPEC = pl.BlockSpec((2 * D, D), lambda i: (0, 0))
_WO_SPEC = pl.BlockSpec((D, D), lambda i: (0, 0))
_B_SPEC = pl.BlockSpec((1, D), lambda i: (0, 0))
_ROW_SPEC = pl.BlockSpec((R, D), lambda i: (i, 0))


def _gates_body(axa, axb, aha, ahb, dga, dgb, h, wr, wu, br, bu, u_o, rh_o):
    dn = 1.0 / jnp.maximum(dga[:, :1] + dgb[:, :1], 1.0)
    ax = (axa[...] + axb[...]) * dn
    ah = (aha[...] + ahb[...]) * dn
    wr_ = wr[...]
    wu_ = wu[...]
    r = jax.nn.sigmoid(jnp.dot(ax, wr_[:D], preferred_element_type=jnp.float32)
                       + jnp.dot(ah, wr_[D:], preferred_element_type=jnp.float32)
                       + br[...])
    u = jax.nn.sigmoid(jnp.dot(ax, wu_[:D], preferred_element_type=jnp.float32)
                       + jnp.dot(ah, wu_[D:], preferred_element_type=jnp.float32)
                       + bu[...])
    u_o[...] = u
    rh_o[...] = r * h[...]


_gates_call = pl.pallas_call(
    _gates_body,
    grid=(NB,),
    in_specs=_part_specs() + _part_specs() + _part_specs()
    + [_ROW_SPEC, _W_SPEC, _W_SPEC, _B_SPEC, _B_SPEC],
    out_specs=[_ROW_SPEC, _ROW_SPEC],
    out_shape=[jax.ShapeDtypeStruct((N, D), jnp.float32),
               jax.ShapeDtypeStruct((N, D), jnp.float32)],
    compiler_params=pltpu.CompilerParams(dimension_semantics=("parallel",)),
)


def _cand_body(axa, axb, aca, acb, dga, dgb, h, u, wc, bc, x_o):
    dn = 1.0 / jnp.maximum(dga[:, :1] + dgb[:, :1], 1.0)
    ax = (axa[...] + axb[...]) * dn
    ac = (aca[...] + acb[...]) * dn
    wc_ = wc[...]
    cand = jnp.tanh(jnp.dot(ax, wc_[:D], preferred_element_type=jnp.float32)
                    + jnp.dot(ac, wc_[D:], preferred_element_type=jnp.float32)
                    + bc[...])
    uu = u[...]
    x_o[...] = uu * h[...] + (1.0 - uu) * cand


_cand_call = pl.pallas_call(
    _cand_body,
    grid=(NB,),
    in_specs=_part_specs() + _part_specs() + _part_specs()
    + [_ROW_SPEC, _ROW_SPEC, _W_SPEC, _B_SPEC],
    out_specs=_ROW_SPEC,
    out_shape=jax.ShapeDtypeStruct((N, D), jnp.float32),
    compiler_params=pltpu.CompilerParams(dimension_semantics=("parallel",)),
)


def _cand_proj_body(axa, axb, aca, acb, dga, dgb, h, u, wc, bc, wo, bo,
                    x_o, out_o):
    dn = 1.0 / jnp.maximum(dga[:, :1] + dgb[:, :1], 1.0)
    ax = (axa[...] + axb[...]) * dn
    ac = (aca[...] + acb[...]) * dn
    wc_ = wc[...]
    cand = jnp.tanh(jnp.dot(ax, wc_[:D], preferred_element_type=jnp.float32)
                    + jnp.dot(ac, wc_[D:], preferred_element_type=jnp.float32)
                    + bc[...])
    uu = u[...]
    x2 = uu * h[...] + (1.0 - uu) * cand
    x_o[...] = x2
    out_o[...] = jnp.dot(x2, wo[...], preferred_element_type=jnp.float32) + bo[...]


_cand_proj_call = pl.pallas_call(
    _cand_proj_body,
    grid=(NB,),
    in_specs=_part_specs() + _part_specs() + _part_specs()
    + [_ROW_SPEC, _ROW_SPEC, _W_SPEC, _B_SPEC, _WO_SPEC, _B_SPEC],
    out_specs=[_ROW_SPEC, _ROW_SPEC],
    out_shape=[jax.ShapeDtypeStruct((N, D), jnp.float32),
               jax.ShapeDtypeStruct((N, D), jnp.float32)],
    compiler_params=pltpu.CompilerParams(dimension_semantics=("parallel",)),
)


def kernel(x, edge_index, h0, h1, Wr0, br0, Wu0, bu0, Wc0, bc0,
           Wr1, br1, Wu1, bu1, Wc1, bc1, Wo, bo):
    src = edge_index[0]
    dst = edge_index[1]
    zfull = jnp.zeros((N, D), jnp.float32)
    ones_h = jnp.ones((CH, D), jnp.float32)

    br0_ = br0.reshape(1, D)
    bu0_ = bu0.reshape(1, D)
    bc0_ = bc0.reshape(1, D)
    br1_ = br1.reshape(1, D)
    bu1_ = bu1.reshape(1, D)
    bc1_ = bc1.reshape(1, D)
    bo_ = bo.reshape(1, D)

    degp = _degsum(dst, zfull, ones_h)
    xagg = _segsum(x, src, dst, zfull)
    h0agg = _segsum(h0, src, dst, zfull)
    u0, rh0 = _gates_call(xagg, xagg, h0agg, h0agg, degp, degp,
                          h0, Wr0, Wu0, br0_, bu0_)
    rh0agg = _segsum(rh0, src, dst, zfull)
    x1 = _cand_call(xagg, xagg, rh0agg, rh0agg, degp, degp,
                    h0, u0, Wc0, bc0_)
    x1agg = _segsum(x1, src, dst, zfull)
    h1agg = _segsum(h1, src, dst, zfull)
    u1, rh1 = _gates_call(x1agg, x1agg, h1agg, h1agg, degp, degp,
                          h1, Wr1, Wu1, br1_, bu1_)
    rh1agg = _segsum(rh1, src, dst, zfull)
    x2, out = _cand_proj_call(x1agg, x1agg, rh1agg, rh1agg, degp, degp,
                              h1, u1, Wc1, bc1_, Wo, bo_)
    return (out, x1, x2)


# speedup vs baseline: 1.7509x; 1.0256x over previous
import jax
import jax.numpy as jnp
from jax import lax
from jax.experimental import pallas as pl
from jax.experimental.pallas import tpu as pltpu
from jax.experimental.pallas import tpu_sc as plsc

N = 10000
E = 320000
D = 128
NC, NS = 2, 16
NW = NC * NS
EPW = E // NW
CH = 80
NCH = EPW // CH
STRIPE = 624
NB = 10
R = N // NB

_mesh = plsc.VectorSubcoreMesh(core_axis_name="c", subcore_axis_name="s")


def _worker_prologue():
    c = lax.axis_index("c")
    s = lax.axis_index("s")
    base0 = (c * NS + s) * EPW
    my = pl.multiple_of(s * STRIPE, 8)

    def stripe_copy(mk_src, mk_dst):
        @pl.when(s < NS - 1)
        def _():
            pltpu.sync_copy(mk_src(STRIPE), mk_dst(STRIPE))

        @pl.when(s == NS - 1)
        def _():
            pltpu.sync_copy(mk_src(640), mk_dst(640))

    return c, s, base0, my, stripe_copy


NBUF = 4


def _segsum_body(feats, srcv, dstv, zfull, out_p,
                 s0, s1, s2, s3, d0, d1, d2, d3, r0, r1, r2, r3, acc,
                 is0, is1, is2, is3, gs0, gs1, gs2, gs3,
                 ds0, ds1, ds2, ds3, ss0, ss1, ss2, ss3):
    c, s, base0, my, stripe_copy = _worker_prologue()
    sbuf = (s0, s1, s2, s3)
    dbuf = (d0, d1, d2, d3)
    rbuf = (r0, r1, r2, r3)
    isem = (is0, is1, is2, is3)
    gsem = (gs0, gs1, gs2, gs3)
    dsem = (ds0, ds1, ds2, ds3)
    ssem = (ss0, ss1, ss2, ss3)

    stripe_copy(lambda n: zfull.at[pl.ds(my, n)],
                lambda n: acc.at[pl.ds(my, n)])
    plsc.subcore_barrier()

    def run_chunks(i0, nk):
        dds, iis = [], []
        for k in range(nk):
            bi = pl.multiple_of(base0 + (i0 + k) * CH, 8)
            dds.append(pltpu.async_copy(dstv.at[pl.ds(bi, CH)],
                                        dbuf[k], dsem[k]))
            iis.append(pltpu.async_copy(srcv.at[pl.ds(bi, CH)],
                                        sbuf[k], isem[k]))
        ggs = []
        for k in range(nk):
            iis[k].wait()
            ggs.append(pltpu.async_copy(feats.at[sbuf[k]], rbuf[k], gsem[k]))
        scats = []
        for k in range(nk):
            dds[k].wait()
            ggs[k].wait()
            scats.append(pltpu.async_copy(rbuf[k], acc.at[dbuf[k]],
                                          ssem[k], add=True))
        for sc_ in scats:
            sc_.wait()

    def body(j, carry):
        run_chunks(j * NBUF, NBUF)
        return carry

    lax.fori_loop(0, NCH // NBUF, body, 0)
    run_chunks(NCH - 1, 1)

    plsc.subcore_barrier()
    dst_row = pl.multiple_of(c * N + s * STRIPE, 8)
    stripe_copy(lambda n: acc.at[pl.ds(my, n)],
                lambda n: out_p.at[pl.ds(dst_row, n)])


_segsum = pl.kernel(
    _segsum_body,
    out_type=jax.ShapeDtypeStruct((NC * N, D), jnp.float32),
    mesh=_mesh,
    scratch_types=(
        (pltpu.VMEM((CH,), jnp.int32),) * (2 * NBUF)
        + (pltpu.VMEM((CH, D), jnp.float32),) * NBUF
        + (pltpu.VMEM_SHARED((N, D), jnp.float32),)
        + (pltpu.SemaphoreType.DMA,) * (4 * NBUF)
    ),
)


DD = 64


def _degsum_body(dstv, zfull, ones_h, out_p,
                 ones_v, d0, d1, d2, d3, acc,
                 ds0, ds1, ds2, ds3, ss0, ss1, ss2, ss3):
    c, s, base0, my, stripe_copy = _worker_prologue()
    dbuf = (d0, d1, d2, d3)
    dsem = (ds0, ds1, ds2, ds3)
    ssem = (ss0, ss1, ss2, ss3)

    stripe_copy(lambda n: zfull.at[pl.ds(my, n)],
                lambda n: acc.at[pl.ds(my, n)])
    pltpu.sync_copy(ones_h, ones_v)
    plsc.subcore_barrier()

    def start(i, k):
        bi = pl.multiple_of(base0 + i * CH, 8)
        return pltpu.async_copy(dstv.at[pl.ds(bi, CH)], dbuf[k], dsem[k])

    def body(j, carry):
        i0 = j * NBUF
        descs = [start(i0 + k, k) for k in range(NBUF)]
        scats = []
        for k in range(NBUF):
            descs[k].wait()
            scats.append(pltpu.async_copy(ones_v, acc.at[dbuf[k]],
                                          ssem[k], add=True))
        for sc_ in scats:
            sc_.wait()
        return carry

    lax.fori_loop(0, NCH // NBUF, body, 0)

    dd = start(NCH - 1, 0)
    dd.wait()
    pltpu.sync_copy(ones_v, acc.at[dbuf[0]], add=True)

    plsc.subcore_barrier()
    dst_row = pl.multiple_of(c * N + s * STRIPE, 8)
    stripe_copy(lambda n: acc.at[pl.ds(my, n)],
                lambda n: out_p.at[pl.ds(dst_row, n)])


_degsum = pl.kernel(
    _degsum_body,
    out_type=jax.ShapeDtypeStruct((NC * N, DD), jnp.float32),
    mesh=_mesh,
    scratch_types=(
        (pltpu.VMEM((CH, DD), jnp.float32),)
        + (pltpu.VMEM((CH,), jnp.int32),) * NBUF
        + (pltpu.VMEM_SHARED((N, DD), jnp.float32),)
        + (pltpu.SemaphoreType.DMA,) * (2 * NBUF)
    ),
    compiler_params=pltpu.CompilerParams(use_tc_tiling_on_sc=False),
)


def _part_specs():
    return [pl.BlockSpec((R, D), lambda i: (i, 0)),
            pl.BlockSpec((R, D), lambda i: (i + NB, 0))]


def _deg_specs():
    return [pl.BlockSpec((R, DD), lambda i: (i, 0)),
            pl.BlockSpec((R, DD), lambda i: (i + NB, 0))]


_W_SPEC = pl.BlockSpec((2 * D, D), lambda i: (0, 0))
_WO_SPEC = pl.BlockSpec((D, D), lambda i: (0, 0))
_B_SPEC = pl.BlockSpec((1, D), lambda i: (0, 0))
_ROW_SPEC = pl.BlockSpec((R, D), lambda i: (i, 0))


def _gates_body(axa, axb, aha, ahb, dga, dgb, h, wr, wu, br, bu, u_o, rh_o):
    dn = 1.0 / jnp.maximum(dga[:, :1] + dgb[:, :1], 1.0)
    ax = (axa[...] + axb[...]) * dn
    ah = (aha[...] + ahb[...]) * dn
    wr_ = wr[...]
    wu_ = wu[...]
    r = jax.nn.sigmoid(jnp.dot(ax, wr_[:D], preferred_element_type=jnp.float32)
                       + jnp.dot(ah, wr_[D:], preferred_element_type=jnp.float32)
                       + br[...])
    u = jax.nn.sigmoid(jnp.dot(ax, wu_[:D], preferred_element_type=jnp.float32)
                       + jnp.dot(ah, wu_[D:], preferred_element_type=jnp.float32)
                       + bu[...])
    u_o[...] = u
    rh_o[...] = r * h[...]


_gates_call = pl.pallas_call(
    _gates_body,
    grid=(NB,),
    in_specs=_part_specs() + _part_specs() + _deg_specs()
    + [_ROW_SPEC, _W_SPEC, _W_SPEC, _B_SPEC, _B_SPEC],
    out_specs=[_ROW_SPEC, _ROW_SPEC],
    out_shape=[jax.ShapeDtypeStruct((N, D), jnp.float32),
               jax.ShapeDtypeStruct((N, D), jnp.float32)],
    compiler_params=pltpu.CompilerParams(dimension_semantics=("parallel",)),
)


def _cand_body(axa, axb, aca, acb, dga, dgb, h, u, wc, bc, x_o):
    dn = 1.0 / jnp.maximum(dga[:, :1] + dgb[:, :1], 1.0)
    ax = (axa[...] + axb[...]) * dn
    ac = (aca[...] + acb[...]) * dn
    wc_ = wc[...]
    cand = jnp.tanh(jnp.dot(ax, wc_[:D], preferred_element_type=jnp.float32)
                    + jnp.dot(ac, wc_[D:], preferred_element_type=jnp.float32)
                    + bc[...])
    uu = u[...]
    x_o[...] = uu * h[...] + (1.0 - uu) * cand


_cand_call = pl.pallas_call(
    _cand_body,
    grid=(NB,),
    in_specs=_part_specs() + _part_specs() + _deg_specs()
    + [_ROW_SPEC, _ROW_SPEC, _W_SPEC, _B_SPEC],
    out_specs=_ROW_SPEC,
    out_shape=jax.ShapeDtypeStruct((N, D), jnp.float32),
    compiler_params=pltpu.CompilerParams(dimension_semantics=("parallel",)),
)


def _cand_proj_body(axa, axb, aca, acb, dga, dgb, h, u, wc, bc, wo, bo,
                    x_o, out_o):
    dn = 1.0 / jnp.maximum(dga[:, :1] + dgb[:, :1], 1.0)
    ax = (axa[...] + axb[...]) * dn
    ac = (aca[...] + acb[...]) * dn
    wc_ = wc[...]
    cand = jnp.tanh(jnp.dot(ax, wc_[:D], preferred_element_type=jnp.float32)
                    + jnp.dot(ac, wc_[D:], preferred_element_type=jnp.float32)
                    + bc[...])
    uu = u[...]
    x2 = uu * h[...] + (1.0 - uu) * cand
    x_o[...] = x2
    out_o[...] = jnp.dot(x2, wo[...], preferred_element_type=jnp.float32) + bo[...]


_cand_proj_call = pl.pallas_call(
    _cand_proj_body,
    grid=(NB,),
    in_specs=_part_specs() + _part_specs() + _deg_specs()
    + [_ROW_SPEC, _ROW_SPEC, _W_SPEC, _B_SPEC, _WO_SPEC, _B_SPEC],
    out_specs=[_ROW_SPEC, _ROW_SPEC],
    out_shape=[jax.ShapeDtypeStruct((N, D), jnp.float32),
               jax.ShapeDtypeStruct((N, D), jnp.float32)],
    compiler_params=pltpu.CompilerParams(dimension_semantics=("parallel",)),
)


def kernel(x, edge_index, h0, h1, Wr0, br0, Wu0, bu0, Wc0, bc0,
           Wr1, br1, Wu1, bu1, Wc1, bc1, Wo, bo):
    src = edge_index[0]
    dst = edge_index[1]
    zfull = jnp.zeros((N, D), jnp.float32)
    zdeg = jnp.zeros((N, DD), jnp.float32)
    ones_h = jnp.ones((CH, DD), jnp.float32)

    br0_ = br0.reshape(1, D)
    bu0_ = bu0.reshape(1, D)
    bc0_ = bc0.reshape(1, D)
    br1_ = br1.reshape(1, D)
    bu1_ = bu1.reshape(1, D)
    bc1_ = bc1.reshape(1, D)
    bo_ = bo.reshape(1, D)

    degp = _degsum(dst, zdeg, ones_h)
    xagg = _segsum(x, src, dst, zfull)
    h0agg = _segsum(h0, src, dst, zfull)
    u0, rh0 = _gates_call(xagg, xagg, h0agg, h0agg, degp, degp,
                          h0, Wr0, Wu0, br0_, bu0_)
    rh0agg = _segsum(rh0, src, dst, zfull)
    x1 = _cand_call(xagg, xagg, rh0agg, rh0agg, degp, degp,
                    h0, u0, Wc0, bc0_)
    x1agg = _segsum(x1, src, dst, zfull)
    h1agg = _segsum(h1, src, dst, zfull)
    u1, rh1 = _gates_call(x1agg, x1agg, h1agg, h1agg, degp, degp,
                          h1, Wr1, Wu1, br1_, bu1_)
    rh1agg = _segsum(rh1, src, dst, zfull)
    x2, out = _cand_proj_call(x1agg, x1agg, rh1agg, rh1agg, degp, degp,
                              h1, u1, Wc1, bc1_, Wo, bo_)
    return (out, x1, x2)


# speedup vs baseline: 1.7760x; 1.0144x over previous
import jax
import jax.numpy as jnp
from jax import lax
from jax.experimental import pallas as pl
from jax.experimental.pallas import tpu as pltpu
from jax.experimental.pallas import tpu_sc as plsc

N = 10000
E = 320000
D = 128
NC, NS = 2, 16
NW = NC * NS
EPW = E // NW
CH = 80
NCH = EPW // CH
STRIPE = 624
NB = 10
R = N // NB

_mesh = plsc.VectorSubcoreMesh(core_axis_name="c", subcore_axis_name="s")


def _worker_prologue():
    c = lax.axis_index("c")
    s = lax.axis_index("s")
    base0 = (c * NS + s) * EPW
    my = pl.multiple_of(s * STRIPE, 8)

    def stripe_copy(mk_src, mk_dst):
        @pl.when(s < NS - 1)
        def _():
            pltpu.sync_copy(mk_src(STRIPE), mk_dst(STRIPE))

        @pl.when(s == NS - 1)
        def _():
            pltpu.sync_copy(mk_src(640), mk_dst(640))

    return c, s, base0, my, stripe_copy


NBUF = 4


def _segsum_body(feats, srcv, dstv, zfull, out_p,
                 s0, s1, s2, s3, d0, d1, d2, d3, r0, r1, r2, r3, acc,
                 is0, is1, is2, is3, gs0, gs1, gs2, gs3,
                 ds0, ds1, ds2, ds3, ss0, ss1, ss2, ss3):
    c, s, base0, my, stripe_copy = _worker_prologue()
    sbuf = (s0, s1, s2, s3)
    dbuf = (d0, d1, d2, d3)
    rbuf = (r0, r1, r2, r3)
    isem = (is0, is1, is2, is3)
    gsem = (gs0, gs1, gs2, gs3)
    dsem = (ds0, ds1, ds2, ds3)
    ssem = (ss0, ss1, ss2, ss3)

    stripe_copy(lambda n: zfull.at[pl.ds(my, n)],
                lambda n: acc.at[pl.ds(my, n)])
    plsc.subcore_barrier()

    def run_chunks(i0, nk):
        dds, iis = [], []
        for k in range(nk):
            bi = pl.multiple_of(base0 + (i0 + k) * CH, 8)
            dds.append(pltpu.async_copy(dstv.at[pl.ds(bi, CH)],
                                        dbuf[k], dsem[k]))
            iis.append(pltpu.async_copy(srcv.at[pl.ds(bi, CH)],
                                        sbuf[k], isem[k]))
        ggs = []
        for k in range(nk):
            iis[k].wait()
            ggs.append(pltpu.async_copy(feats.at[sbuf[k]], rbuf[k], gsem[k]))
        scats = []
        for k in range(nk):
            dds[k].wait()
            ggs[k].wait()
            scats.append(pltpu.async_copy(rbuf[k], acc.at[dbuf[k]],
                                          ssem[k], add=True))
        for sc_ in scats:
            sc_.wait()

    def body(j, carry):
        run_chunks(j * NBUF, NBUF)
        return carry

    lax.fori_loop(0, NCH // NBUF, body, 0)
    run_chunks(NCH - 1, 1)

    plsc.subcore_barrier()
    dst_row = pl.multiple_of(c * N + s * STRIPE, 8)
    stripe_copy(lambda n: acc.at[pl.ds(my, n)],
                lambda n: out_p.at[pl.ds(dst_row, n)])


_segsum = pl.kernel(
    _segsum_body,
    out_type=jax.ShapeDtypeStruct((NC * N, D), jnp.float32),
    mesh=_mesh,
    scratch_types=(
        (pltpu.VMEM((CH,), jnp.int32),) * (2 * NBUF)
        + (pltpu.VMEM((CH, D), jnp.float32),) * NBUF
        + (pltpu.VMEM_SHARED((N, D), jnp.float32),)
        + (pltpu.SemaphoreType.DMA,) * (4 * NBUF)
    ),
)


DD = 32


def _degsum_body(dstv, zfull, ones_h, out_p,
                 ones_v, d0, d1, d2, d3, acc,
                 ds0, ds1, ds2, ds3, ss0, ss1, ss2, ss3):
    c, s, base0, my, stripe_copy = _worker_prologue()
    dbuf = (d0, d1, d2, d3)
    dsem = (ds0, ds1, ds2, ds3)
    ssem = (ss0, ss1, ss2, ss3)

    stripe_copy(lambda n: zfull.at[pl.ds(my, n)],
                lambda n: acc.at[pl.ds(my, n)])
    pltpu.sync_copy(ones_h, ones_v)
    plsc.subcore_barrier()

    def start(i, k):
        bi = pl.multiple_of(base0 + i * CH, 8)
        return pltpu.async_copy(dstv.at[pl.ds(bi, CH)], dbuf[k], dsem[k])

    def body(j, carry):
        i0 = j * NBUF
        descs = [start(i0 + k, k) for k in range(NBUF)]
        scats = []
        for k in range(NBUF):
            descs[k].wait()
            scats.append(pltpu.async_copy(ones_v, acc.at[dbuf[k]],
                                          ssem[k], add=True))
        for sc_ in scats:
            sc_.wait()
        return carry

    lax.fori_loop(0, NCH // NBUF, body, 0)

    dd = start(NCH - 1, 0)
    dd.wait()
    pltpu.sync_copy(ones_v, acc.at[dbuf[0]], add=True)

    plsc.subcore_barrier()
    dst_row = pl.multiple_of(c * N + s * STRIPE, 8)
    stripe_copy(lambda n: acc.at[pl.ds(my, n)],
                lambda n: out_p.at[pl.ds(dst_row, n)])


_degsum = pl.kernel(
    _degsum_body,
    out_type=jax.ShapeDtypeStruct((NC * N, DD), jnp.float32),
    mesh=_mesh,
    scratch_types=(
        (pltpu.VMEM((CH, DD), jnp.float32),)
        + (pltpu.VMEM((CH,), jnp.int32),) * NBUF
        + (pltpu.VMEM_SHARED((N, DD), jnp.float32),)
        + (pltpu.SemaphoreType.DMA,) * (2 * NBUF)
    ),
    compiler_params=pltpu.CompilerParams(use_tc_tiling_on_sc=False),
)


def _part_specs():
    return [pl.BlockSpec((R, D), lambda i: (i, 0)),
            pl.BlockSpec((R, D), lambda i: (i + NB, 0))]


def _deg_specs():
    return [pl.BlockSpec((R, DD), lambda i: (i, 0)),
            pl.BlockSpec((R, DD), lambda i: (i + NB, 0))]


_W_SPEC = pl.BlockSpec((2 * D, D), lambda i: (0, 0))
_WO_SPEC = pl.BlockSpec((D, D), lambda i: (0, 0))
_B_SPEC = pl.BlockSpec((1, D), lambda i: (0, 0))
_ROW_SPEC = pl.BlockSpec((R, D), lambda i: (i, 0))


def _gates_body(axa, axb, aha, ahb, dga, dgb, h, wr, wu, br, bu, u_o, rh_o):
    dn = 1.0 / jnp.maximum(dga[:, :1] + dgb[:, :1], 1.0)
    ax = (axa[...] + axb[...]) * dn
    ah = (aha[...] + ahb[...]) * dn
    wr_ = wr[...]
    wu_ = wu[...]
    r = jax.nn.sigmoid(jnp.dot(ax, wr_[:D], preferred_element_type=jnp.float32)
                       + jnp.dot(ah, wr_[D:], preferred_element_type=jnp.float32)
                       + br[...])
    u = jax.nn.sigmoid(jnp.dot(ax, wu_[:D], preferred_element_type=jnp.float32)
                       + jnp.dot(ah, wu_[D:], preferred_element_type=jnp.float32)
                       + bu[...])
    u_o[...] = u
    rh_o[...] = r * h[...]


_gates_call = pl.pallas_call(
    _gates_body,
    grid=(NB,),
    in_specs=_part_specs() + _part_specs() + _deg_specs()
    + [_ROW_SPEC, _W_SPEC, _W_SPEC, _B_SPEC, _B_SPEC],
    out_specs=[_ROW_SPEC, _ROW_SPEC],
    out_shape=[jax.ShapeDtypeStruct((N, D), jnp.float32),
               jax.ShapeDtypeStruct((N, D), jnp.float32)],
    compiler_params=pltpu.CompilerParams(dimension_semantics=("parallel",)),
)


def _cand_body(axa, axb, aca, acb, dga, dgb, h, u, wc, bc, x_o):
    dn = 1.0 / jnp.maximum(dga[:, :1] + dgb[:, :1], 1.0)
    ax = (axa[...] + axb[...]) * dn
    ac = (aca[...] + acb[...]) * dn
    wc_ = wc[...]
    cand = jnp.tanh(jnp.dot(ax, wc_[:D], preferred_element_type=jnp.float32)
                    + jnp.dot(ac, wc_[D:], preferred_element_type=jnp.float32)
                    + bc[...])
    uu = u[...]
    x_o[...] = uu * h[...] + (1.0 - uu) * cand


_cand_call = pl.pallas_call(
    _cand_body,
    grid=(NB,),
    in_specs=_part_specs() + _part_specs() + _deg_specs()
    + [_ROW_SPEC, _ROW_SPEC, _W_SPEC, _B_SPEC],
    out_specs=_ROW_SPEC,
    out_shape=jax.ShapeDtypeStruct((N, D), jnp.float32),
    compiler_params=pltpu.CompilerParams(dimension_semantics=("parallel",)),
)


def _cand_proj_body(axa, axb, aca, acb, dga, dgb, h, u, wc, bc, wo, bo,
                    x_o, out_o):
    dn = 1.0 / jnp.maximum(dga[:, :1] + dgb[:, :1], 1.0)
    ax = (axa[...] + axb[...]) * dn
    ac = (aca[...] + acb[...]) * dn
    wc_ = wc[...]
    cand = jnp.tanh(jnp.dot(ax, wc_[:D], preferred_element_type=jnp.float32)
                    + jnp.dot(ac, wc_[D:], preferred_element_type=jnp.float32)
                    + bc[...])
    uu = u[...]
    x2 = uu * h[...] + (1.0 - uu) * cand
    x_o[...] = x2
    out_o[...] = jnp.dot(x2, wo[...], preferred_element_type=jnp.float32) + bo[...]


_cand_proj_call = pl.pallas_call(
    _cand_proj_body,
    grid=(NB,),
    in_specs=_part_specs() + _part_specs() + _deg_specs()
    + [_ROW_SPEC, _ROW_SPEC, _W_SPEC, _B_SPEC, _WO_SPEC, _B_SPEC],
    out_specs=[_ROW_SPEC, _ROW_SPEC],
    out_shape=[jax.ShapeDtypeStruct((N, D), jnp.float32),
               jax.ShapeDtypeStruct((N, D), jnp.float32)],
    compiler_params=pltpu.CompilerParams(dimension_semantics=("parallel",)),
)


def kernel(x, edge_index, h0, h1, Wr0, br0, Wu0, bu0, Wc0, bc0,
           Wr1, br1, Wu1, bu1, Wc1, bc1, Wo, bo):
    src = edge_index[0]
    dst = edge_index[1]
    zfull = jnp.zeros((N, D), jnp.float32)
    zdeg = jnp.zeros((N, DD), jnp.float32)
    ones_h = jnp.ones((CH, DD), jnp.float32)

    br0_ = br0.reshape(1, D)
    bu0_ = bu0.reshape(1, D)
    bc0_ = bc0.reshape(1, D)
    br1_ = br1.reshape(1, D)
    bu1_ = bu1.reshape(1, D)
    bc1_ = bc1.reshape(1, D)
    bo_ = bo.reshape(1, D)

    degp = _degsum(dst, zdeg, ones_h)
    xagg = _segsum(x, src, dst, zfull)
    h0agg = _segsum(h0, src, dst, zfull)
    u0, rh0 = _gates_call(xagg, xagg, h0agg, h0agg, degp, degp,
                          h0, Wr0, Wu0, br0_, bu0_)
    rh0agg = _segsum(rh0, src, dst, zfull)
    x1 = _cand_call(xagg, xagg, rh0agg, rh0agg, degp, degp,
                    h0, u0, Wc0, bc0_)
    x1agg = _segsum(x1, src, dst, zfull)
    h1agg = _segsum(h1, src, dst, zfull)
    u1, rh1 = _gates_call(x1agg, x1agg, h1agg, h1agg, degp, degp,
                          h1, Wr1, Wu1, br1_, bu1_)
    rh1agg = _segsum(rh1, src, dst, zfull)
    x2, out = _cand_proj_call(x1agg, x1agg, rh1agg, rh1agg, degp, degp,
                              h1, u1, Wc1, bc1_, Wo, bo_)
    return (out, x1, x2)


# speedup vs baseline: 1.7897x; 1.0077x over previous
import jax
import jax.numpy as jnp
from jax import lax
from jax.experimental import pallas as pl
from jax.experimental.pallas import tpu as pltpu
from jax.experimental.pallas import tpu_sc as plsc

N = 10000
E = 320000
D = 128
NC, NS = 2, 16
NW = NC * NS
EPW = E // NW
CH = 80
NCH = EPW // CH
STRIPE = 624
NB = 10
R = N // NB

_mesh = plsc.VectorSubcoreMesh(core_axis_name="c", subcore_axis_name="s")


def _worker_prologue():
    c = lax.axis_index("c")
    s = lax.axis_index("s")
    base0 = (c * NS + s) * EPW
    my = pl.multiple_of(s * STRIPE, 8)

    def stripe_copy(mk_src, mk_dst):
        @pl.when(s < NS - 1)
        def _():
            pltpu.sync_copy(mk_src(STRIPE), mk_dst(STRIPE))

        @pl.when(s == NS - 1)
        def _():
            pltpu.sync_copy(mk_src(640), mk_dst(640))

    return c, s, base0, my, stripe_copy


NBUF = 4


def _segsum_body(feats, srcv, dstv, zfull, out_p,
                 s0, s1, s2, s3, d0, d1, d2, d3, r0, r1, r2, r3, acc,
                 is0, is1, is2, is3, gs0, gs1, gs2, gs3,
                 ds0, ds1, ds2, ds3, ss0, ss1, ss2, ss3):
    c, s, base0, my, stripe_copy = _worker_prologue()
    sbuf = (s0, s1, s2, s3)
    dbuf = (d0, d1, d2, d3)
    rbuf = (r0, r1, r2, r3)
    isem = (is0, is1, is2, is3)
    gsem = (gs0, gs1, gs2, gs3)
    dsem = (ds0, ds1, ds2, ds3)
    ssem = (ss0, ss1, ss2, ss3)

    stripe_copy(lambda n: zfull.at[pl.ds(my, n)],
                lambda n: acc.at[pl.ds(my, n)])
    plsc.subcore_barrier()

    def run_chunks(i0, nk):
        dds, iis = [], []
        for k in range(nk):
            bi = pl.multiple_of(base0 + (i0 + k) * CH, 8)
            dds.append(pltpu.async_copy(dstv.at[pl.ds(bi, CH)],
                                        dbuf[k], dsem[k]))
            iis.append(pltpu.async_copy(srcv.at[pl.ds(bi, CH)],
                                        sbuf[k], isem[k]))
        ggs = []
        for k in range(nk):
            iis[k].wait()
            ggs.append(pltpu.async_copy(feats.at[sbuf[k]], rbuf[k], gsem[k]))
        scats = []
        for k in range(nk):
            dds[k].wait()
            ggs[k].wait()
            scats.append(pltpu.async_copy(rbuf[k], acc.at[dbuf[k]],
                                          ssem[k], add=True))
        for sc_ in scats:
            sc_.wait()

    def body(j, carry):
        run_chunks(j * NBUF, NBUF)
        return carry

    lax.fori_loop(0, NCH // NBUF, body, 0)
    run_chunks(NCH - 1, 1)

    plsc.subcore_barrier()
    dst_row = pl.multiple_of(c * N + s * STRIPE, 8)
    stripe_copy(lambda n: acc.at[pl.ds(my, n)],
                lambda n: out_p.at[pl.ds(dst_row, n)])


_segsum = pl.kernel(
    _segsum_body,
    out_type=jax.ShapeDtypeStruct((NC * N, D), jnp.float32),
    mesh=_mesh,
    scratch_types=(
        (pltpu.VMEM((CH,), jnp.int32),) * (2 * NBUF)
        + (pltpu.VMEM((CH, D), jnp.float32),) * NBUF
        + (pltpu.VMEM_SHARED((N, D), jnp.float32),)
        + (pltpu.SemaphoreType.DMA,) * (4 * NBUF)
    ),
)


DD = 16


def _degsum_body(dstv, zfull, ones_h, out_p,
                 ones_v, d0, d1, d2, d3, acc,
                 ds0, ds1, ds2, ds3, ss0, ss1, ss2, ss3):
    c, s, base0, my, stripe_copy = _worker_prologue()
    dbuf = (d0, d1, d2, d3)
    dsem = (ds0, ds1, ds2, ds3)
    ssem = (ss0, ss1, ss2, ss3)

    stripe_copy(lambda n: zfull.at[pl.ds(my, n)],
                lambda n: acc.at[pl.ds(my, n)])
    pltpu.sync_copy(ones_h, ones_v)
    plsc.subcore_barrier()

    def start(i, k):
        bi = pl.multiple_of(base0 + i * CH, 8)
        return pltpu.async_copy(dstv.at[pl.ds(bi, CH)], dbuf[k], dsem[k])

    def body(j, carry):
        i0 = j * NBUF
        descs = [start(i0 + k, k) for k in range(NBUF)]
        scats = []
        for k in range(NBUF):
            descs[k].wait()
            scats.append(pltpu.async_copy(ones_v, acc.at[dbuf[k]],
                                          ssem[k], add=True))
        for sc_ in scats:
            sc_.wait()
        return carry

    lax.fori_loop(0, NCH // NBUF, body, 0)

    dd = start(NCH - 1, 0)
    dd.wait()
    pltpu.sync_copy(ones_v, acc.at[dbuf[0]], add=True)

    plsc.subcore_barrier()
    dst_row = pl.multiple_of(c * N + s * STRIPE, 8)
    stripe_copy(lambda n: acc.at[pl.ds(my, n)],
                lambda n: out_p.at[pl.ds(dst_row, n)])


_degsum = pl.kernel(
    _degsum_body,
    out_type=jax.ShapeDtypeStruct((NC * N, DD), jnp.float32),
    mesh=_mesh,
    scratch_types=(
        (pltpu.VMEM((CH, DD), jnp.float32),)
        + (pltpu.VMEM((CH,), jnp.int32),) * NBUF
        + (pltpu.VMEM_SHARED((N, DD), jnp.float32),)
        + (pltpu.SemaphoreType.DMA,) * (2 * NBUF)
    ),
    compiler_params=pltpu.CompilerParams(use_tc_tiling_on_sc=False),
)


def _part_specs():
    return [pl.BlockSpec((R, D), lambda i: (i, 0)),
            pl.BlockSpec((R, D), lambda i: (i + NB, 0))]


def _deg_specs():
    return [pl.BlockSpec((R, DD), lambda i: (i, 0)),
            pl.BlockSpec((R, DD), lambda i: (i + NB, 0))]


_W_SPEC = pl.BlockSpec((2 * D, D), lambda i: (0, 0))
_WO_SPEC = pl.BlockSpec((D, D), lambda i: (0, 0))
_B_SPEC = pl.BlockSpec((1, D), lambda i: (0, 0))
_ROW_SPEC = pl.BlockSpec((R, D), lambda i: (i, 0))


def _gates_body(axa, axb, aha, ahb, dga, dgb, h, wr, wu, br, bu, u_o, rh_o):
    dn = 1.0 / jnp.maximum(dga[:, :1] + dgb[:, :1], 1.0)
    ax = (axa[...] + axb[...]) * dn
    ah = (aha[...] + ahb[...]) * dn
    wr_ = wr[...]
    wu_ = wu[...]
    r = jax.nn.sigmoid(jnp.dot(ax, wr_[:D], preferred_element_type=jnp.float32)
                       + jnp.dot(ah, wr_[D:], preferred_element_type=jnp.float32)
                       + br[...])
    u = jax.nn.sigmoid(jnp.dot(ax, wu_[:D], preferred_element_type=jnp.float32)
                       + jnp.dot(ah, wu_[D:], preferred_element_type=jnp.float32)
                       + bu[...])
    u_o[...] = u
    rh_o[...] = r * h[...]


_gates_call = pl.pallas_call(
    _gates_body,
    grid=(NB,),
    in_specs=_part_specs() + _part_specs() + _deg_specs()
    + [_ROW_SPEC, _W_SPEC, _W_SPEC, _B_SPEC, _B_SPEC],
    out_specs=[_ROW_SPEC, _ROW_SPEC],
    out_shape=[jax.ShapeDtypeStruct((N, D), jnp.float32),
               jax.ShapeDtypeStruct((N, D), jnp.float32)],
    compiler_params=pltpu.CompilerParams(dimension_semantics=("parallel",)),
)


def _cand_body(axa, axb, aca, acb, dga, dgb, h, u, wc, bc, x_o):
    dn = 1.0 / jnp.maximum(dga[:, :1] + dgb[:, :1], 1.0)
    ax = (axa[...] + axb[...]) * dn
    ac = (aca[...] + acb[...]) * dn
    wc_ = wc[...]
    cand = jnp.tanh(jnp.dot(ax, wc_[:D], preferred_element_type=jnp.float32)
                    + jnp.dot(ac, wc_[D:], preferred_element_type=jnp.float32)
                    + bc[...])
    uu = u[...]
    x_o[...] = uu * h[...] + (1.0 - uu) * cand


_cand_call = pl.pallas_call(
    _cand_body,
    grid=(NB,),
    in_specs=_part_specs() + _part_specs() + _deg_specs()
    + [_ROW_SPEC, _ROW_SPEC, _W_SPEC, _B_SPEC],
    out_specs=_ROW_SPEC,
    out_shape=jax.ShapeDtypeStruct((N, D), jnp.float32),
    compiler_params=pltpu.CompilerParams(dimension_semantics=("parallel",)),
)


def _cand_proj_body(axa, axb, aca, acb, dga, dgb, h, u, wc, bc, wo, bo,
                    x_o, out_o):
    dn = 1.0 / jnp.maximum(dga[:, :1] + dgb[:, :1], 1.0)
    ax = (axa[...] + axb[...]) * dn
    ac = (aca[...] + acb[...]) * dn
    wc_ = wc[...]
    cand = jnp.tanh(jnp.dot(ax, wc_[:D], preferred_element_type=jnp.float32)
                    + jnp.dot(ac, wc_[D:], preferred_element_type=jnp.float32)
                    + bc[...])
    uu = u[...]
    x2 = uu * h[...] + (1.0 - uu) * cand
    x_o[...] = x2
    out_o[...] = jnp.dot(x2, wo[...], preferred_element_type=jnp.float32) + bo[...]


_cand_proj_call = pl.pallas_call(
    _cand_proj_body,
    grid=(NB,),
    in_specs=_part_specs() + _part_specs() + _deg_specs()
    + [_ROW_SPEC, _ROW_SPEC, _W_SPEC, _B_SPEC, _WO_SPEC, _B_SPEC],
    out_specs=[_ROW_SPEC, _ROW_SPEC],
    out_shape=[jax.ShapeDtypeStruct((N, D), jnp.float32),
               jax.ShapeDtypeStruct((N, D), jnp.float32)],
    compiler_params=pltpu.CompilerParams(dimension_semantics=("parallel",)),
)


def kernel(x, edge_index, h0, h1, Wr0, br0, Wu0, bu0, Wc0, bc0,
           Wr1, br1, Wu1, bu1, Wc1, bc1, Wo, bo):
    src = edge_index[0]
    dst = edge_index[1]
    zfull = jnp.zeros((N, D), jnp.float32)
    zdeg = jnp.zeros((N, DD), jnp.float32)
    ones_h = jnp.ones((CH, DD), jnp.float32)

    br0_ = br0.reshape(1, D)
    bu0_ = bu0.reshape(1, D)
    bc0_ = bc0.reshape(1, D)
    br1_ = br1.reshape(1, D)
    bu1_ = bu1.reshape(1, D)
    bc1_ = bc1.reshape(1, D)
    bo_ = bo.reshape(1, D)

    degp = _degsum(dst, zdeg, ones_h)
    xagg = _segsum(x, src, dst, zfull)
    h0agg = _segsum(h0, src, dst, zfull)
    u0, rh0 = _gates_call(xagg, xagg, h0agg, h0agg, degp, degp,
                          h0, Wr0, Wu0, br0_, bu0_)
    rh0agg = _segsum(rh0, src, dst, zfull)
    x1 = _cand_call(xagg, xagg, rh0agg, rh0agg, degp, degp,
                    h0, u0, Wc0, bc0_)
    x1agg = _segsum(x1, src, dst, zfull)
    h1agg = _segsum(h1, src, dst, zfull)
    u1, rh1 = _gates_call(x1agg, x1agg, h1agg, h1agg, degp, degp,
                          h1, Wr1, Wu1, br1_, bu1_)
    rh1agg = _segsum(rh1, src, dst, zfull)
    x2, out = _cand_proj_call(x1agg, x1agg, rh1agg, rh1agg, degp, degp,
                              h1, u1, Wc1, bc1_, Wo, bo_)
    return (out, x1, x2)
